# baseline (speedup 1.0000x reference)
import functools

import jax
import jax.numpy as jnp
from jax import lax
from jax.experimental import pallas as pl
from jax.experimental.pallas import tpu as pltpu
from jax.experimental.pallas import tpu_sc as plsc

NUM_BINS = 64
F = 32
N = 1048576
NC, NS, L = 2, 16, 16
NW = NC * NS
ROWS_W = N // NW
WORDS_W = ROWS_W * F
CHUNK_ROWS = 1024
CHUNK_W = CHUNK_ROWS * F
NCHUNK = ROWS_W // CHUNK_ROWS
UNROLL = 8
HIST_W = NUM_BINS * F


def _minmax_body(x_ref, min_ref, max_ref):
    i = pl.program_id(0)
    bmin = jnp.min(x_ref[...], axis=0, keepdims=True)
    bmax = jnp.max(x_ref[...], axis=0, keepdims=True)

    @pl.when(i == 0)
    def _():
        min_ref[...] = bmin
        max_ref[...] = bmax

    @pl.when(i > 0)
    def _():
        min_ref[...] = jnp.minimum(min_ref[...], bmin)
        max_ref[...] = jnp.maximum(max_ref[...], bmax)


def _colwise_minmax(x128):
    rows = x128.shape[0]
    block_rows = 2048
    grid = rows // block_rows
    mn, mx = pl.pallas_call(
        _minmax_body,
        grid=(grid,),
        in_specs=[pl.BlockSpec((block_rows, 128), lambda i: (i, 0))],
        out_specs=[
            pl.BlockSpec((1, 128), lambda i: (0, 0)),
            pl.BlockSpec((1, 128), lambda i: (0, 0)),
        ],
        out_shape=[
            jax.ShapeDtypeStruct((1, 128), jnp.float32),
            jax.ShapeDtypeStruct((1, 128), jnp.float32),
        ],
    )(x128)
    return mn, mx


def _sc_hist_body(x_hbm, invw_hbm, bias2_hbm, out_hbm, buf0, buf1, cvec,
                  hist_a, hist_b, sem0, sem1):
    buf = [buf0, buf1]
    sem = [sem0, sem1]
    wid = lax.axis_index("s") * NC + lax.axis_index("c")
    base = wid * WORDS_W

    pltpu.sync_copy(invw_hbm, cvec.at[pl.ds(0, F)])
    pltpu.sync_copy(bias2_hbm, cvec.at[pl.ds(F, F)])
    invw_g = [cvec[pl.ds(0, L)], cvec[pl.ds(L, L)]]
    bias2_g = [cvec[pl.ds(F, L)], cvec[pl.ds(F + L, L)]]
    iota = lax.iota(jnp.int32, L)
    lo_g = [iota * NUM_BINS, (iota + L) * NUM_BINS]
    hi_g = [lo_g[0] + (NUM_BINS - 1), lo_g[1] + (NUM_BINS - 1)]
    ones = jnp.full((L,), 1.0, jnp.float32)
    zeros = jnp.zeros((L,), jnp.float32)
    hist_g = [hist_a, hist_b]

    def zero_body(i, _):
        hist_a[pl.ds(i * L, L)] = zeros
        hist_b[pl.ds(i * L, L)] = zeros
        return 0

    lax.fori_loop(0, HIST_W // L, zero_body, 0)

    def compute(b):
        @plsc.parallel_loop(0, CHUNK_W // (2 * L), unroll=UNROLL)
        def vec_body(j):
            w0 = j * (2 * L)
            for g in range(2):
                x = buf[b][pl.ds(w0 + g * L, L)]
                t = x * invw_g[g] + bias2_g[g]
                idx = t.astype(jnp.int32)
                idx = jnp.minimum(jnp.maximum(idx, lo_g[g]), hi_g[g])
                plsc.addupdate_scatter(hist_g[g], [idx], ones)

    def start(ch, b):
        pltpu.async_copy(
            x_hbm.at[pl.ds(base + ch * CHUNK_W, CHUNK_W)], buf[b], sem[b])

    def wait(b):
        pltpu.make_async_copy(x_hbm.at[pl.ds(0, CHUNK_W)], buf[b],
                              sem[b]).wait()

    start(0, 0)

    @pl.loop(0, NCHUNK, step=2)
    def chunk_body(ch):
        wait(0)
        start(ch + 1, 1)
        compute(0)
        wait(1)

        @pl.when(ch + 2 < NCHUNK)
        def _():
            start(ch + 2, 0)

        compute(1)

    def merge_body(i, _):
        s = pl.ds(i * L, L)
        hist_a[s] = hist_a[s] + hist_b[s]
        return 0

    lax.fori_loop(0, HIST_W // L, merge_body, 0)
    pltpu.sync_copy(hist_a, out_hbm.at[wid])


def _sc_hist(x_flat, invw, bias2):
    mesh = plsc.VectorSubcoreMesh(
        core_axis_name="c", subcore_axis_name="s", num_cores=NC,
        num_subcores=NS)
    run = pl.kernel(
        _sc_hist_body,
        out_type=jax.ShapeDtypeStruct((NW, HIST_W), jnp.float32),
        mesh=mesh,
        scratch_types=[
            pltpu.VMEM((CHUNK_W,), jnp.float32),
            pltpu.VMEM((CHUNK_W,), jnp.float32),
            pltpu.VMEM((2 * F,), jnp.float32),
            pltpu.VMEM((HIST_W,), jnp.float32),
            pltpu.VMEM((HIST_W,), jnp.float32),
            pltpu.SemaphoreType.DMA,
            pltpu.SemaphoreType.DMA,
        ],
        compiler_params=pltpu.CompilerParams(
            needs_layout_passes=False, use_tc_tiling_on_sc=True),
    )
    return run(x_flat, invw, bias2)


@jax.jit
def kernel(inputs):
    x128 = inputs.reshape(N // 4, 128)
    mn, mx = _colwise_minmax(x128)
    col_min = jnp.min(mn.reshape(4, F), axis=0)
    col_max = jnp.max(mx.reshape(4, F), axis=0)
    mins = col_min - 0.5
    maxs = col_max + 0.5
    width = (maxs - mins) / NUM_BINS
    invw = 1.0 / width
    col = jnp.arange(F, dtype=jnp.float32)
    bias2 = -mins * invw + col * NUM_BINS

    tiles = _sc_hist(x128.reshape(-1), invw, bias2)
    counts = tiles.sum(axis=0).reshape(F, NUM_BINS)
    return counts.T


# speedup vs baseline: 1.4799x; 1.4799x over previous
import functools

import jax
import jax.numpy as jnp
from jax import lax
from jax.experimental import pallas as pl
from jax.experimental.pallas import tpu as pltpu
from jax.experimental.pallas import tpu_sc as plsc

NUM_BINS = 64
F = 32
N = 1048576
NC, NS, L = 2, 16, 16
NW = NC * NS
NROW128 = N // 4
ROWS_W = NROW128 // NW
CHUNK_ROWS = 256
CHUNK_W = CHUNK_ROWS * 128
NCHUNK = ROWS_W // CHUNK_ROWS
UNROLL = 8
HIST_W = NUM_BINS * F


def _minmax_body(x_ref, min_ref, max_ref):
    i = pl.program_id(0)
    bmin = jnp.min(x_ref[...], axis=0, keepdims=True)
    bmax = jnp.max(x_ref[...], axis=0, keepdims=True)

    @pl.when(i == 0)
    def _():
        min_ref[...] = bmin
        max_ref[...] = bmax

    @pl.when(i > 0)
    def _():
        min_ref[...] = jnp.minimum(min_ref[...], bmin)
        max_ref[...] = jnp.maximum(max_ref[...], bmax)


def _colwise_minmax(x128):
    rows = x128.shape[0]
    block_rows = 2048
    grid = rows // block_rows
    mn, mx = pl.pallas_call(
        _minmax_body,
        grid=(grid,),
        in_specs=[pl.BlockSpec((block_rows, 128), lambda i: (i, 0))],
        out_specs=[
            pl.BlockSpec((1, 128), lambda i: (0, 0)),
            pl.BlockSpec((1, 128), lambda i: (0, 0)),
        ],
        out_shape=[
            jax.ShapeDtypeStruct((1, 128), jnp.float32),
            jax.ShapeDtypeStruct((1, 128), jnp.float32),
        ],
    )(x128)
    return mn, mx


def _sc_hist_body(x_hbm, invw_hbm, bias2_hbm, out_hbm, buf0, buf1, cvec,
                  hist_a, hist_b, sem0, sem1):
    buf = [buf0, buf1]
    sem = [sem0, sem1]
    wid = lax.axis_index("s") * NC + lax.axis_index("c")
    base = wid * ROWS_W

    pltpu.sync_copy(invw_hbm, cvec.at[pl.ds(0, F)])
    pltpu.sync_copy(bias2_hbm, cvec.at[pl.ds(F, F)])
    invw_g = [cvec[pl.ds(0, L)], cvec[pl.ds(L, L)]]
    bias2_g = [cvec[pl.ds(F, L)], cvec[pl.ds(F + L, L)]]
    iota = lax.iota(jnp.int32, L)
    lo_g = [iota * NUM_BINS, (iota + L) * NUM_BINS]
    hi_g = [lo_g[0] + (NUM_BINS - 1), lo_g[1] + (NUM_BINS - 1)]
    ones = jnp.full((L,), 1.0, jnp.float32)
    zeros = jnp.zeros((L,), jnp.float32)
    hist_g = [hist_a, hist_b]

    def zero_body(i, _):
        hist_a[pl.ds(i * L, L)] = zeros
        hist_b[pl.ds(i * L, L)] = zeros
        return 0

    lax.fori_loop(0, HIST_W // L, zero_body, 0)

    def compute(b):
        @plsc.parallel_loop(0, CHUNK_ROWS, unroll=2)
        def vec_body(r):
            for k in range(8):
                g = k % 2
                x = buf[b][r, pl.ds(k * L, L)]
                t = x * invw_g[g] + bias2_g[g]
                idx = t.astype(jnp.int32)
                idx = jnp.minimum(jnp.maximum(idx, lo_g[g]), hi_g[g])
                plsc.addupdate_scatter(hist_g[g], [idx], ones)

    def start(ch, b):
        pltpu.async_copy(
            x_hbm.at[pl.ds(base + ch * CHUNK_ROWS, CHUNK_ROWS)], buf[b],
            sem[b])

    def wait(b):
        pltpu.make_async_copy(x_hbm.at[pl.ds(0, CHUNK_ROWS)], buf[b],
                              sem[b]).wait()

    start(0, 0)

    @pl.loop(0, NCHUNK, step=2)
    def chunk_body(ch):
        wait(0)
        start(ch + 1, 1)
        compute(0)
        wait(1)

        @pl.when(ch + 2 < NCHUNK)
        def _():
            start(ch + 2, 0)

        compute(1)

    def merge_body(i, _):
        s = pl.ds(i * L, L)
        hist_a[s] = hist_a[s] + hist_b[s]
        return 0

    lax.fori_loop(0, HIST_W // L, merge_body, 0)
    pltpu.sync_copy(hist_a, out_hbm.at[wid])


def _sc_hist(x_flat, invw, bias2):
    mesh = plsc.VectorSubcoreMesh(
        core_axis_name="c", subcore_axis_name="s", num_cores=NC,
        num_subcores=NS)
    run = pl.kernel(
        _sc_hist_body,
        out_type=jax.ShapeDtypeStruct((NW, HIST_W), jnp.float32),
        mesh=mesh,
        scratch_types=[
            pltpu.VMEM((CHUNK_ROWS, 128), jnp.float32),
            pltpu.VMEM((CHUNK_ROWS, 128), jnp.float32),
            pltpu.VMEM((2 * F,), jnp.float32),
            pltpu.VMEM((HIST_W,), jnp.float32),
            pltpu.VMEM((HIST_W,), jnp.float32),
            pltpu.SemaphoreType.DMA,
            pltpu.SemaphoreType.DMA,
        ],
        compiler_params=pltpu.CompilerParams(
            needs_layout_passes=False, use_tc_tiling_on_sc=True),
    )
    return run(x_flat, invw, bias2)


@jax.jit
def kernel(inputs):
    x128 = inputs.reshape(N // 4, 128)
    mn, mx = _colwise_minmax(x128)
    col_min = jnp.min(mn.reshape(4, F), axis=0)
    col_max = jnp.max(mx.reshape(4, F), axis=0)
    mins = col_min - 0.5
    maxs = col_max + 0.5
    width = (maxs - mins) / NUM_BINS
    invw = 1.0 / width
    col = jnp.arange(F, dtype=jnp.float32)
    bias2 = -mins * invw + col * NUM_BINS

    tiles = _sc_hist(x128, invw, bias2)
    counts = tiles.sum(axis=0).reshape(F, NUM_BINS)
    return counts.T
